# Initial kernel scaffold; baseline (speedup 1.0000x reference)
import jax
import jax.numpy as jnp
from jax.experimental import pallas as pl


def kernel(logits, emb_weight):
    raise NotImplementedError("write your pallas kernel here")


# speedup vs baseline: 11.9653x; 11.9653x over previous
import functools

import jax
import jax.numpy as jnp
from jax import lax
from jax.experimental import pallas as pl
from jax.experimental.pallas import tpu as pltpu
from jax.experimental.pallas import tpu_sc as plsc

B = 64
V = 128000
D = 2048
K = 50

NB = 1024
CH = 6400
NCH = V // CH
VREGS = CH // 16
CAND = 2048
GH = 32
NEG_INF = float("-inf")
I32_MAX = 2**31 - 1


def _body(logits, emb, out, buf0, buf1, hist, cand_v, cand_i,
          sel_v, sel_i, wts, rows, acc, sem0, sem1, gsem):
    num_cores = 2
    wid = lax.axis_index("s") * num_cores + lax.axis_index("c")
    lane = lax.iota(jnp.int32, 16)
    ones = jnp.ones((16,), jnp.int32)
    zeros_f = jnp.zeros((16,), jnp.float32)
    ninf = jnp.full((16,), NEG_INF, jnp.float32)
    lane0 = lane == 0

    def stream_row(row, process, carry0):
        rbase = row * V
        pltpu.make_async_copy(
            logits.at[pl.ds(rbase, CH)], buf0, sem0).start()

        def do_chunk(c, cur_buf, cur_sem, nxt_buf, nxt_sem, carry):
            @pl.when(c + 1 < NCH)
            def _():
                pltpu.make_async_copy(
                    logits.at[pl.ds(rbase + (c + 1) * CH, CH)],
                    nxt_buf, nxt_sem).start()
            pltpu.make_async_copy(
                logits.at[pl.ds(rbase + c * CH, CH)], cur_buf, cur_sem).wait()
            return process(carry, cur_buf, c)

        def pair(c2, carry):
            carry = do_chunk(2 * c2, buf0, sem0, buf1, sem1, carry)
            carry = do_chunk(2 * c2 + 1, buf1, sem1, buf0, sem0, carry)
            return carry

        return lax.fori_loop(0, NCH // 2, pair, carry0)

    def run_row(r_local, _):
        row = wid * 2 + r_local

        def zero_hist(i, c):
            hist[pl.ds(i * 16, 16)] = jnp.zeros((16,), jnp.int32)
            return c
        lax.fori_loop(0, NB, zero_hist, 0)

        def hist_chunk(carry, buf, c):
            def one(i, cc):
                x = buf[pl.ds(i * 16, 16)]
                bits = lax.bitcast_convert_type(x, jnp.int32)
                m = bits ^ (lax.shift_right_arithmetic(bits, 31)
                            | jnp.int32(-(2**31)))
                bucket = lax.shift_right_logical(m, 22)
                addr = lax.shift_left(bucket, 4) + lane
                plsc.addupdate_scatter(hist, [addr], ones)
                return cc
            return lax.fori_loop(0, VREGS, one, carry)
        stream_row(row, hist_chunk, 0)

        def scan_bucket(t, carry):
            cum, tb = carry
            b = NB - 1 - t
            cnt = jnp.sum(hist[pl.ds(b * 16, 16)])
            cum2 = cum + cnt
            tb = jnp.where((cum < K) & (cum2 >= K), b, tb)
            return (cum2, tb)
        _, tb = lax.fori_loop(0, NB, scan_bucket,
                              (jnp.int32(0), jnp.int32(0)))
        tbits = jnp.where(tb >= NB // 2,
                          lax.shift_left(tb - NB // 2, 22),
                          ~lax.shift_left(tb, 22))
        t_vec = lax.bitcast_convert_type(jnp.full((16,), tbits, jnp.int32), jnp.float32)

        def filt_chunk(off, buf, c):
            base = c * CH

            def one(i, off):
                x = buf[pl.ds(i * 16, 16)]
                mask = x >= t_vec
                ivec = jnp.full((16,), base + i * 16, jnp.int32) + lane
                plsc.store_compressed(cand_v.at[pl.ds(off, 16)], x, mask=mask)
                plsc.store_compressed(cand_i.at[pl.ds(off, 16)], ivec,
                                      mask=mask)
                cnt = jnp.max(plsc.all_reduce_population_count(mask))
                return jnp.minimum(off + cnt, CAND)
            return lax.fori_loop(0, VREGS, one, off)
        off = stream_row(row, filt_chunk, jnp.int32(0))

        cand_v[pl.ds(off, 16)] = ninf
        nv = lax.shift_right_logical(off + 15, 4)

        for j in range(4):
            sel_v[pl.ds(j * 16, 16)] = ninf
            sel_i[pl.ds(j * 16, 16)] = jnp.full((16,), j * 16, jnp.int32) + lane

        def pick(k, _):
            def vmax(j, best):
                return jnp.maximum(best, cand_v[pl.ds(j * 16, 16)])
            m = jnp.max(lax.fori_loop(0, nv, vmax, ninf))
            m_spl = jnp.full((16,), m, jnp.float32)

            def vsel(j, selv):
                eq = cand_v[pl.ds(j * 16, 16)] == m_spl
                return jnp.minimum(
                    selv, jnp.where(eq, cand_i[pl.ds(j * 16, 16)],
                                    jnp.full((16,), I32_MAX, jnp.int32)))
            sel = jnp.min(lax.fori_loop(
                0, nv, vsel, jnp.full((16,), I32_MAX, jnp.int32)))
            sel_spl = jnp.full((16,), sel, jnp.int32)

            def vclr(j, c):
                v = cand_v[pl.ds(j * 16, 16)]
                iv = cand_i[pl.ds(j * 16, 16)]
                kill = (v == m_spl) & (iv == sel_spl)
                cand_v[pl.ds(j * 16, 16)] = jnp.where(kill, ninf, v)
                return c
            lax.fori_loop(0, nv, vclr, 0)

            kk = jnp.full((16,), k, jnp.int32)
            plsc.store_scatter(sel_v, [kk], m_spl, mask=lane0)
            plsc.store_scatter(sel_i, [kk], sel_spl, mask=lane0)
            return 0
        lax.fori_loop(0, K, pick, 0)

        v0 = sel_v[pl.ds(0, 16)]
        v1 = sel_v[pl.ds(16, 16)]
        v2 = sel_v[pl.ds(32, 16)]
        v3 = sel_v[pl.ds(48, 16)]
        mmax = jnp.max(jnp.maximum(jnp.maximum(v0, v1),
                                   jnp.maximum(v2, v3)))
        m_spl = jnp.full((16,), mmax, jnp.float32)
        e0 = jnp.exp(v0 - m_spl)
        e1 = jnp.exp(v1 - m_spl)
        e2 = jnp.exp(v2 - m_spl)
        e3 = jnp.exp(v3 - m_spl)
        s = jnp.sum(e0 + e1 + e2 + e3)
        s_spl = jnp.full((16,), s, jnp.float32)
        wts[pl.ds(0, 16)] = e0 / s_spl
        wts[pl.ds(16, 16)] = e1 / s_spl
        wts[pl.ds(32, 16)] = e2 / s_spl
        wts[pl.ds(48, 16)] = e3 / s_spl

        def zero_acc(j, c):
            acc[pl.ds(j * 16, 16)] = zeros_f
            return c
        lax.fori_loop(0, D // 16, zero_acc, 0)

        for h in range(64 // GH):
            pltpu.async_copy(
                emb.at[sel_i.at[pl.ds(h * GH, GH)]], rows, gsem).wait()

            def mix_row(r, c):
                w_spl = plsc.load_gather(
                    wts, [jnp.full((16,), h * GH + r, jnp.int32)])

                def fma(j, cc):
                    plsc.addupdate(
                        acc.at[pl.ds(j * 16, 16)],
                        w_spl * rows[r, pl.ds(j * 16, 16)])
                    return cc
                return lax.fori_loop(0, D // 16, fma, c)
            lax.fori_loop(0, GH, mix_row, 0)

        pltpu.sync_copy(acc, out.at[pl.ds(row * D, D)])
        return 0

    lax.fori_loop(0, 2, run_row, 0)


@jax.jit
def kernel(logits, emb_weight):
    mesh = plsc.VectorSubcoreMesh(core_axis_name="c", subcore_axis_name="s")
    k = functools.partial(
        pl.kernel,
        out_type=jax.ShapeDtypeStruct((B * D,), jnp.float32),
        mesh=mesh,
        compiler_params=pltpu.CompilerParams(needs_layout_passes=False),
        scratch_types=[
            pltpu.VMEM((CH,), jnp.float32),
            pltpu.VMEM((CH,), jnp.float32),
            pltpu.VMEM((NB * 16,), jnp.int32),
            pltpu.VMEM((CAND + 16,), jnp.float32),
            pltpu.VMEM((CAND + 16,), jnp.int32),
            pltpu.VMEM((64,), jnp.float32),
            pltpu.VMEM((64,), jnp.int32),
            pltpu.VMEM((64,), jnp.float32),
            pltpu.VMEM((GH, D), jnp.float32),
            pltpu.VMEM((D,), jnp.float32),
            pltpu.SemaphoreType.DMA,
            pltpu.SemaphoreType.DMA,
            pltpu.SemaphoreType.DMA,
        ],
    )(_body)
    return k(logits.reshape(B * V), emb_weight).reshape(B, D)


# speedup vs baseline: 15.8068x; 1.3211x over previous
import functools

import jax
import jax.numpy as jnp
from jax import lax
from jax.experimental import pallas as pl
from jax.experimental.pallas import tpu as pltpu
from jax.experimental.pallas import tpu_sc as plsc

B = 64
V = 128000
D = 2048
K = 50

NB = 1024
CH = 6400
NCH = V // CH
VREGS = CH // 16
GRP = 16
NMAX = V // 16 // GRP
CAND = 2048
GH = 16
NEG_INF = float("-inf")
I32_MAX = 2**31 - 1


def _body(logits, emb, out, buf0, buf1, maxima, hist, cand_v, cand_i,
          sel_v, sel_i, wts, rows0, rows1, acc, sem0, sem1, gsem0, gsem1):
    num_cores = 2
    wid = lax.axis_index("s") * num_cores + lax.axis_index("c")
    lane = lax.iota(jnp.int32, 16)
    ones = jnp.ones((16,), jnp.int32)
    zeros_f = jnp.zeros((16,), jnp.float32)
    ninf = jnp.full((16,), NEG_INF, jnp.float32)
    lane0 = lane == 0

    def to_bucket(x):
        bits = lax.bitcast_convert_type(x, jnp.int32)
        m = bits ^ (lax.shift_right_arithmetic(bits, 31)
                    | jnp.int32(-(2**31)))
        return lax.shift_right_logical(m, 22)

    def stream_row(row, process, carry0):
        rbase = row * V
        pltpu.make_async_copy(
            logits.at[pl.ds(rbase, CH)], buf0, sem0).start()

        def do_chunk(c, cur_buf, cur_sem, nxt_buf, nxt_sem, carry):
            @pl.when(c + 1 < NCH)
            def _():
                pltpu.make_async_copy(
                    logits.at[pl.ds(rbase + (c + 1) * CH, CH)],
                    nxt_buf, nxt_sem).start()
            pltpu.make_async_copy(
                logits.at[pl.ds(rbase + c * CH, CH)], cur_buf, cur_sem).wait()
            return process(carry, cur_buf, c)

        def pair(c2, carry):
            carry = do_chunk(2 * c2, buf0, sem0, buf1, sem1, carry)
            carry = do_chunk(2 * c2 + 1, buf1, sem1, buf0, sem0, carry)
            return carry

        return lax.fori_loop(0, NCH // 2, pair, carry0)

    def run_row(r_local, _):
        row = wid * 2 + r_local

        def max_chunk(gmax, buf, c):
            def grp16(g, gm):
                m = buf[pl.ds(g * GRP * 16, 16)]
                for u in range(1, GRP):
                    m = jnp.maximum(m, buf[pl.ds((g * GRP + u) * 16, 16)])
                maxima[pl.ds(c * (VREGS // GRP) * 16 + g * 16, 16)] = m
                return jnp.maximum(gm, m)
            return lax.fori_loop(0, VREGS // GRP, grp16, gmax)
        gmax = stream_row(row, max_chunk, ninf)

        def zero_hist(i, c):
            for u in range(8):
                hist[pl.ds((i * 8 + u) * 16, 16)] = jnp.zeros((16,), jnp.int32)
            return c
        lax.fori_loop(0, NB // 8, zero_hist, 0)

        def hist_vreg(i, c):
            addr = lax.shift_left(to_bucket(maxima[pl.ds(i * 16, 16)]),
                                  4) + lane
            plsc.addupdate_scatter(hist, [addr], ones)
            return c
        lax.fori_loop(0, NMAX, hist_vreg, 0)

        sb = jnp.max(to_bucket(gmax))

        def scan_cond(carry):
            b, cum = carry
            return (cum < K) & (b >= 0)

        def scan_body(carry):
            b, cum = carry
            return (b - 1, cum + jnp.sum(hist[pl.ds(b * 16, 16)]))
        bend, _ = lax.while_loop(scan_cond, scan_body, (sb, jnp.int32(0)))
        tb = bend + 1
        tbits = jnp.where(tb >= NB // 2,
                          lax.shift_left(tb - NB // 2, 22),
                          ~lax.shift_left(tb, 22))
        t_vec = lax.bitcast_convert_type(
            jnp.full((16,), tbits, jnp.int32), jnp.float32)

        def filt_chunk(off, buf, c):
            base = c * CH

            def quad(q, off):
                for u in range(4):
                    i = q * 4 + u
                    x = buf[pl.ds(i * 16, 16)]
                    mask = x >= t_vec
                    ivec = jnp.full((16,), base + i * 16, jnp.int32) + lane
                    plsc.store_compressed(cand_v.at[pl.ds(off, 16)], x,
                                          mask=mask)
                    plsc.store_compressed(cand_i.at[pl.ds(off, 16)], ivec,
                                          mask=mask)
                    cnt = jnp.max(plsc.all_reduce_population_count(mask))
                    off = jnp.minimum(off + cnt, CAND)
                return off
            return lax.fori_loop(0, VREGS // 4, quad, off)
        off = stream_row(row, filt_chunk, jnp.int32(0))

        cand_v[pl.ds(off, 16)] = ninf
        nv = lax.shift_right_logical(off + 15, 4)

        for j in range(4):
            sel_v[pl.ds(j * 16, 16)] = ninf
            sel_i[pl.ds(j * 16, 16)] = jnp.full((16,), j * 16, jnp.int32) + lane

        def pick(k, _):
            def vmax(j, best):
                return jnp.maximum(best, cand_v[pl.ds(j * 16, 16)])
            m = jnp.max(lax.fori_loop(0, nv, vmax, ninf))
            m_spl = jnp.full((16,), m, jnp.float32)

            def vsel(j, selv):
                eq = cand_v[pl.ds(j * 16, 16)] == m_spl
                return jnp.minimum(
                    selv, jnp.where(eq, cand_i[pl.ds(j * 16, 16)],
                                    jnp.full((16,), I32_MAX, jnp.int32)))
            sel = jnp.min(lax.fori_loop(
                0, nv, vsel, jnp.full((16,), I32_MAX, jnp.int32)))
            sel_spl = jnp.full((16,), sel, jnp.int32)

            def vclr(j, c):
                v = cand_v[pl.ds(j * 16, 16)]
                iv = cand_i[pl.ds(j * 16, 16)]
                kill = (v == m_spl) & (iv == sel_spl)
                cand_v[pl.ds(j * 16, 16)] = jnp.where(kill, ninf, v)
                return c
            lax.fori_loop(0, nv, vclr, 0)

            kk = jnp.full((16,), k, jnp.int32)
            plsc.store_scatter(sel_v, [kk], m_spl, mask=lane0)
            plsc.store_scatter(sel_i, [kk], sel_spl, mask=lane0)
            return 0
        lax.fori_loop(0, K, pick, 0)

        v0 = sel_v[pl.ds(0, 16)]
        v1 = sel_v[pl.ds(16, 16)]
        v2 = sel_v[pl.ds(32, 16)]
        v3 = sel_v[pl.ds(48, 16)]
        mmax = jnp.max(jnp.maximum(jnp.maximum(v0, v1),
                                   jnp.maximum(v2, v3)))
        m_spl = jnp.full((16,), mmax, jnp.float32)
        e0 = jnp.exp(v0 - m_spl)
        e1 = jnp.exp(v1 - m_spl)
        e2 = jnp.exp(v2 - m_spl)
        e3 = jnp.exp(v3 - m_spl)
        s = jnp.sum(e0 + e1 + e2 + e3)
        s_spl = jnp.full((16,), s, jnp.float32)
        wts[pl.ds(0, 16)] = e0 / s_spl
        wts[pl.ds(16, 16)] = e1 / s_spl
        wts[pl.ds(32, 16)] = e2 / s_spl
        wts[pl.ds(48, 16)] = e3 / s_spl

        def zero_acc(j, c):
            for u in range(8):
                acc[pl.ds((j * 8 + u) * 16, 16)] = zeros_f
            return c
        lax.fori_loop(0, D // 128, zero_acc, 0)

        nbat = 64 // GH
        bufs = (rows0, rows1)
        sems = (gsem0, gsem1)
        for g in range(nbat):
            rows, gsem = bufs[g % 2], sems[g % 2]
            pltpu.async_copy(
                emb.at[sel_i.at[pl.ds(g * GH, GH)]], rows, gsem).wait()

            def mix_row(r, c):
                w_spl = plsc.load_gather(
                    wts, [jnp.full((16,), g * GH + r, jnp.int32)])

                def fma(jo, cc):
                    for u in range(8):
                        j = jo * 8 + u
                        plsc.addupdate(
                            acc.at[pl.ds(j * 16, 16)],
                            w_spl * rows[r, pl.ds(j * 16, 16)])
                    return cc
                return lax.fori_loop(0, D // 128, fma, c)
            lax.fori_loop(0, GH, mix_row, 0)

        pltpu.sync_copy(acc, out.at[pl.ds(row * D, D)])
        return 0

    lax.fori_loop(0, 2, run_row, 0)


@jax.jit
def kernel(logits, emb_weight):
    mesh = plsc.VectorSubcoreMesh(core_axis_name="c", subcore_axis_name="s")
    k = functools.partial(
        pl.kernel,
        out_type=jax.ShapeDtypeStruct((B * D,), jnp.float32),
        mesh=mesh,
        compiler_params=pltpu.CompilerParams(needs_layout_passes=False),
        scratch_types=[
            pltpu.VMEM((CH,), jnp.float32),
            pltpu.VMEM((CH,), jnp.float32),
            pltpu.VMEM((V // 16,), jnp.float32),
            pltpu.VMEM((NB * 16,), jnp.int32),
            pltpu.VMEM((CAND + 16,), jnp.float32),
            pltpu.VMEM((CAND + 16,), jnp.int32),
            pltpu.VMEM((64,), jnp.float32),
            pltpu.VMEM((64,), jnp.int32),
            pltpu.VMEM((64,), jnp.float32),
            pltpu.VMEM((GH, D), jnp.float32),
            pltpu.VMEM((GH, D), jnp.float32),
            pltpu.VMEM((D,), jnp.float32),
            pltpu.SemaphoreType.DMA,
            pltpu.SemaphoreType.DMA,
            pltpu.SemaphoreType.DMA,
            pltpu.SemaphoreType.DMA,
        ],
    )(_body)
    return k(logits.reshape(B * V), emb_weight).reshape(B, D)


# speedup vs baseline: 23.7787x; 1.5043x over previous
import functools

import jax
import jax.numpy as jnp
from jax import lax
from jax.experimental import pallas as pl
from jax.experimental.pallas import tpu as pltpu
from jax.experimental.pallas import tpu_sc as plsc

B = 64
V = 128000
D = 2048
K = 50

NB = 1024
CH = 6400
NCH = V // CH
VREGS = CH // 16
GRP = 16
NMAX = V // 16 // GRP
CAND = 2048
HB = 32
GH = 16
NEG_INF = float("-inf")
I32_MAX = 2**31 - 1


def _body(logits, emb, out, buf0, buf1, maxima, hist, cand_v, cand_i,
          sel_v, sel_i, wts, rows0, rows1, acc, hbbuf, blist,
          sem0, sem1, gsem0, gsem1, bsem):
    num_cores = 2
    wid = lax.axis_index("s") * num_cores + lax.axis_index("c")
    lane = lax.iota(jnp.int32, 16)
    ones = jnp.ones((16,), jnp.int32)
    zeros_f = jnp.zeros((16,), jnp.float32)
    ninf = jnp.full((16,), NEG_INF, jnp.float32)
    lane0 = lane == 0

    def to_bucket(x):
        bits = lax.bitcast_convert_type(x, jnp.int32)
        m = bits ^ (lax.shift_right_arithmetic(bits, 31)
                    | jnp.int32(-(2**31)))
        return lax.shift_right_logical(m, 22)

    def stream_row(row, process, carry0):
        rbase = row * V
        pltpu.make_async_copy(
            logits.at[pl.ds(rbase, CH)], buf0, sem0).start()

        def do_chunk(c, cur_buf, cur_sem, nxt_buf, nxt_sem, carry):
            @pl.when(c + 1 < NCH)
            def _():
                pltpu.make_async_copy(
                    logits.at[pl.ds(rbase + (c + 1) * CH, CH)],
                    nxt_buf, nxt_sem).start()
            pltpu.make_async_copy(
                logits.at[pl.ds(rbase + c * CH, CH)], cur_buf, cur_sem).wait()
            return process(carry, cur_buf, c)

        def pair(c2, carry):
            carry = do_chunk(2 * c2, buf0, sem0, buf1, sem1, carry)
            carry = do_chunk(2 * c2 + 1, buf1, sem1, buf0, sem0, carry)
            return carry

        return lax.fori_loop(0, NCH // 2, pair, carry0)

    def run_row(r_local, _):
        row = wid * 2 + r_local

        def max_chunk(gmax, buf, c):
            def grp16(g, gm):
                m = buf[pl.ds(g * GRP * 16, 16)]
                for u in range(1, GRP):
                    m = jnp.maximum(m, buf[pl.ds((g * GRP + u) * 16, 16)])
                maxima[pl.ds(c * (VREGS // GRP) * 16 + g * 16, 16)] = m
                return jnp.maximum(gm, m)
            return lax.fori_loop(0, VREGS // GRP, grp16, gmax)
        gmax = stream_row(row, max_chunk, ninf)

        def zero_hist(i, c):
            for u in range(8):
                hist[pl.ds((i * 8 + u) * 16, 16)] = jnp.zeros((16,), jnp.int32)
            return c
        lax.fori_loop(0, NB // 8, zero_hist, 0)

        def hist_vreg(i, c):
            addr = lax.shift_left(to_bucket(maxima[pl.ds(i * 16, 16)]),
                                  4) + lane
            plsc.addupdate_scatter(hist, [addr], ones)
            return c
        lax.fori_loop(0, NMAX, hist_vreg, 0)

        sb = jnp.max(to_bucket(gmax))

        def scan_cond(carry):
            b, cum = carry
            return (cum < K) & (b >= 0)

        def scan_body(carry):
            b, cum = carry
            return (b - 1, cum + jnp.sum(hist[pl.ds(b * 16, 16)]))
        bend, _ = lax.while_loop(scan_cond, scan_body, (sb, jnp.int32(0)))
        tb = bend + 1
        tbits = jnp.where(tb >= NB // 2,
                          lax.shift_left(tb - NB // 2, 22),
                          ~lax.shift_left(tb, 22))
        t_vec = lax.bitcast_convert_type(
            jnp.full((16,), tbits, jnp.int32), jnp.float32)

        def blkscan(i, hc):
            m = maxima[pl.ds(i * 16, 16)]
            hit = jnp.max(plsc.all_reduce_population_count(m >= t_vec))
            blist[hc] = i
            return hc + jnp.minimum(hit, 1)
        nhit = lax.fori_loop(0, NMAX, blkscan, jnp.int32(0))

        rbase = row * V

        def hchunk(cidx, off):
            base_h = cidx * HB

            def fire(q, c):
                @pl.when(base_h + q < nhit)
                def _():
                    bid = blist[base_h + q]
                    pltpu.make_async_copy(
                        logits.at[pl.ds(rbase + bid * 256, 256)],
                        hbbuf.at[q], bsem).start()
                return c
            lax.fori_loop(0, HB, fire, 0)

            def drain(q, c):
                @pl.when(base_h + q < nhit)
                def _():
                    pltpu.make_async_copy(
                        logits.at[pl.ds(rbase, 256)],
                        hbbuf.at[q], bsem).wait()
                return c
            lax.fori_loop(0, HB, drain, 0)

            def proc(q, off):
                def do(off):
                    bid = blist[base_h + q]

                    def one(i, off):
                        x = hbbuf[q, pl.ds(i * 16, 16)]
                        mask = x >= t_vec
                        ivec = jnp.full((16,), bid * 256 + i * 16,
                                        jnp.int32) + lane
                        plsc.store_compressed(cand_v.at[pl.ds(off, 16)], x,
                                              mask=mask)
                        plsc.store_compressed(cand_i.at[pl.ds(off, 16)], ivec,
                                              mask=mask)
                        cnt = jnp.max(plsc.all_reduce_population_count(mask))
                        return jnp.minimum(off + cnt, CAND)
                    return lax.fori_loop(0, 16, one, off)
                return lax.cond(base_h + q < nhit, do, lambda o: o, off)
            return lax.fori_loop(0, HB, proc, off)

        nhc = lax.shift_right_logical(nhit + HB - 1, 5)
        off = lax.fori_loop(0, nhc, hchunk, jnp.int32(0))

        cand_v[pl.ds(off, 16)] = ninf
        nv = lax.shift_right_logical(off + 15, 4)

        for j in range(4):
            sel_v[pl.ds(j * 16, 16)] = ninf
            sel_i[pl.ds(j * 16, 16)] = jnp.full((16,), j * 16, jnp.int32) + lane

        def pick(k, _):
            def vmax(j, best):
                return jnp.maximum(best, cand_v[pl.ds(j * 16, 16)])
            m = jnp.max(lax.fori_loop(0, nv, vmax, ninf))
            m_spl = jnp.full((16,), m, jnp.float32)

            def vsel(j, selv):
                eq = cand_v[pl.ds(j * 16, 16)] == m_spl
                return jnp.minimum(
                    selv, jnp.where(eq, cand_i[pl.ds(j * 16, 16)],
                                    jnp.full((16,), I32_MAX, jnp.int32)))
            sel = jnp.min(lax.fori_loop(
                0, nv, vsel, jnp.full((16,), I32_MAX, jnp.int32)))
            sel_spl = jnp.full((16,), sel, jnp.int32)

            def vclr(j, c):
                v = cand_v[pl.ds(j * 16, 16)]
                iv = cand_i[pl.ds(j * 16, 16)]
                kill = (v == m_spl) & (iv == sel_spl)
                cand_v[pl.ds(j * 16, 16)] = jnp.where(kill, ninf, v)
                return c
            lax.fori_loop(0, nv, vclr, 0)

            kk = jnp.full((16,), k, jnp.int32)
            plsc.store_scatter(sel_v, [kk], m_spl, mask=lane0)
            plsc.store_scatter(sel_i, [kk], sel_spl, mask=lane0)
            return 0
        lax.fori_loop(0, K, pick, 0)

        v0 = sel_v[pl.ds(0, 16)]
        v1 = sel_v[pl.ds(16, 16)]
        v2 = sel_v[pl.ds(32, 16)]
        v3 = sel_v[pl.ds(48, 16)]
        mmax = jnp.max(jnp.maximum(jnp.maximum(v0, v1),
                                   jnp.maximum(v2, v3)))
        m_spl = jnp.full((16,), mmax, jnp.float32)
        e0 = jnp.exp(v0 - m_spl)
        e1 = jnp.exp(v1 - m_spl)
        e2 = jnp.exp(v2 - m_spl)
        e3 = jnp.exp(v3 - m_spl)
        s = jnp.sum(e0 + e1 + e2 + e3)
        s_spl = jnp.full((16,), s, jnp.float32)
        wts[pl.ds(0, 16)] = e0 / s_spl
        wts[pl.ds(16, 16)] = e1 / s_spl
        wts[pl.ds(32, 16)] = e2 / s_spl
        wts[pl.ds(48, 16)] = e3 / s_spl

        def zero_acc(j, c):
            for u in range(8):
                acc[pl.ds((j * 8 + u) * 16, 16)] = zeros_f
            return c
        lax.fori_loop(0, D // 128, zero_acc, 0)

        nbat = 64 // GH
        bufs = (rows0, rows1)
        sems = (gsem0, gsem1)
        descs = [None] * nbat
        descs[0] = pltpu.async_copy(
            emb.at[sel_i.at[pl.ds(0, GH)]], rows0, gsem0)
        for g in range(nbat):
            rows = bufs[g % 2]
            if g + 1 < nbat:
                descs[g + 1] = pltpu.async_copy(
                    emb.at[sel_i.at[pl.ds((g + 1) * GH, GH)]],
                    bufs[(g + 1) % 2], sems[(g + 1) % 2])
            descs[g].wait()

            def mix_row(r, c):
                w_spl = plsc.load_gather(
                    wts, [jnp.full((16,), g * GH + r, jnp.int32)])

                def fma(jo, cc):
                    for u in range(8):
                        j = jo * 8 + u
                        plsc.addupdate(
                            acc.at[pl.ds(j * 16, 16)],
                            w_spl * rows[r, pl.ds(j * 16, 16)])
                    return cc
                return lax.fori_loop(0, D // 128, fma, c)
            lax.fori_loop(0, GH, mix_row, 0)

        pltpu.sync_copy(acc, out.at[pl.ds(row * D, D)])
        return 0

    lax.fori_loop(0, 2, run_row, 0)


@jax.jit
def kernel(logits, emb_weight):
    mesh = plsc.VectorSubcoreMesh(core_axis_name="c", subcore_axis_name="s")
    k = functools.partial(
        pl.kernel,
        out_type=jax.ShapeDtypeStruct((B * D,), jnp.float32),
        mesh=mesh,
        compiler_params=pltpu.CompilerParams(needs_layout_passes=False),
        scratch_types=[
            pltpu.VMEM((CH,), jnp.float32),
            pltpu.VMEM((CH,), jnp.float32),
            pltpu.VMEM((V // 16,), jnp.float32),
            pltpu.VMEM((NB * 16,), jnp.int32),
            pltpu.VMEM((CAND + 16,), jnp.float32),
            pltpu.VMEM((CAND + 16,), jnp.int32),
            pltpu.VMEM((64,), jnp.float32),
            pltpu.VMEM((64,), jnp.int32),
            pltpu.VMEM((64,), jnp.float32),
            pltpu.VMEM((GH, D), jnp.float32),
            pltpu.VMEM((GH, D), jnp.float32),
            pltpu.VMEM((D,), jnp.float32),
            pltpu.VMEM((HB, 256), jnp.float32),
            pltpu.SMEM((512,), jnp.int32),
            pltpu.SemaphoreType.DMA,
            pltpu.SemaphoreType.DMA,
            pltpu.SemaphoreType.DMA,
            pltpu.SemaphoreType.DMA,
            pltpu.SemaphoreType.DMA,
        ],
    )(_body)
    return k(logits.reshape(B * V), emb_weight).reshape(B, D)
